# baseline (speedup 1.0000x reference)
import functools

import jax
import jax.numpy as jnp
from jax import lax
from jax.experimental import pallas as pl
from jax.experimental.pallas import tpu as pltpu
from jax.experimental.pallas import tpu_sc as plsc

VOCAB = 1000000
CTX = 200
DIM = 64
B = 1024
L = 200

NUM_CORES = 2
NUM_SUBCORES = 16
NW = NUM_CORES * NUM_SUBCORES
BATCHES_PER_W = B // NW
HALF = L // 2


def _sc_body(x_hbm, word_hbm, pos_hbm, out_hbm, idx_v, dest_v, sem):
    wid = lax.axis_index("s") * NUM_CORES + lax.axis_index("c")
    base = wid * BATCHES_PER_W

    pltpu.sync_copy(x_hbm.at[pl.ds(base, BATCHES_PER_W)], idx_v)

    def one_batch(g, carry):
        gb = base + g
        pltpu.sync_copy(pos_hbm, dest_v)
        cp0 = pltpu.async_copy(
            word_hbm.at[idx_v.at[g, 0]], dest_v.at[pl.ds(0, HALF)], sem,
            add=True)
        cp1 = pltpu.async_copy(
            word_hbm.at[idx_v.at[g, 1]], dest_v.at[pl.ds(HALF, HALF)], sem,
            add=True)
        cp0.wait()
        cp1.wait()
        pltpu.sync_copy(dest_v, out_hbm.at[gb])
        return carry

    lax.fori_loop(0, BATCHES_PER_W, one_batch, 0)


@jax.jit
def _embed(x_r, w_word, w_pos):
    mesh = plsc.VectorSubcoreMesh(core_axis_name="c", subcore_axis_name="s")
    run = pl.kernel(
        _sc_body,
        out_type=jax.ShapeDtypeStruct((B, L, DIM), jnp.float32),
        mesh=mesh,
        scratch_types=[
            pltpu.VMEM((BATCHES_PER_W, 2, HALF), jnp.int32),
            pltpu.VMEM((L, DIM), jnp.float32),
            pltpu.SemaphoreType.DMA,
        ],
        compiler_params=pltpu.CompilerParams(use_tc_tiling_on_sc=False),
    )
    return run(x_r, w_word, w_pos)


def kernel(x, W_word, W_pos):
    x_r = x.astype(jnp.int32).reshape(B, 2, HALF)
    return _embed(x_r, W_word, W_pos)


# speedup vs baseline: 1.0435x; 1.0435x over previous
import jax
import jax.numpy as jnp
from jax import lax
from jax.experimental import pallas as pl
from jax.experimental.pallas import tpu as pltpu
from jax.experimental.pallas import tpu_sc as plsc

VOCAB = 1000000
CTX = 200
DIM = 64
B = 1024
L = 200

NUM_CORES = 2
NUM_SUBCORES = 16
NW = NUM_CORES * NUM_SUBCORES
BATCHES_PER_W = B // NW
HALF = L // 2
CB = 2
CHUNKS = BATCHES_PER_W // CB
CL = CB * L


def _sc_body(x_hbm, word_hbm, pos2_hbm, out_hbm,
             idx_v, dest0, dest1, gsem0, gsem1, wsem0, wsem1, psem0, psem1):
    wid = lax.axis_index("s") * NUM_CORES + lax.axis_index("c")
    base = wid * BATCHES_PER_W

    pltpu.sync_copy(x_hbm.at[pl.ds(base, BATCHES_PER_W)], idx_v)

    dests = (dest0, dest1)
    gsems = (gsem0, gsem1)
    wsems = (wsem0, wsem1)
    psems = (psem0, psem1)
    gathers = {}
    writes = {}

    def issue_gathers(c, dest, gsem):
        ds = []
        for b in range(CB):
            for j in range(2):
                ds.append(pltpu.async_copy(
                    word_hbm.at[idx_v.at[CB * c + b, j]],
                    dest.at[pl.ds(b * L + j * HALF, HALF)],
                    gsem, add=True))
        return ds

    def issue_write(c, dest, wsem):
        return pltpu.async_copy(
            dest, out_hbm.at[pl.ds((base + CB * c) * L, CL)], wsem)

    for c in range(CHUNKS):
        s = c % 2
        if c >= 2:
            writes[c - 2].wait()
        init = pltpu.async_copy(pos2_hbm, dests[s], psems[s])
        if c >= 1:
            for d in gathers[c - 1]:
                d.wait()
            writes[c - 1] = issue_write(c - 1, dests[1 - s], wsems[1 - s])
        init.wait()
        gathers[c] = issue_gathers(c, dests[s], gsems[s])

    last = CHUNKS - 1
    for d in gathers[last]:
        d.wait()
    writes[last] = issue_write(last, dests[last % 2], wsems[last % 2])
    writes[last - 1].wait()
    writes[last].wait()


@jax.jit
def _embed(x_r, w_word, w_pos):
    mesh = plsc.VectorSubcoreMesh(core_axis_name="c", subcore_axis_name="s")
    run = pl.kernel(
        _sc_body,
        out_type=jax.ShapeDtypeStruct((B * L, DIM), jnp.float32),
        mesh=mesh,
        scratch_types=[
            pltpu.VMEM((BATCHES_PER_W, 2, HALF), jnp.int32),
            pltpu.VMEM((CL, DIM), jnp.float32),
            pltpu.VMEM((CL, DIM), jnp.float32),
            pltpu.SemaphoreType.DMA,
            pltpu.SemaphoreType.DMA,
            pltpu.SemaphoreType.DMA,
            pltpu.SemaphoreType.DMA,
            pltpu.SemaphoreType.DMA,
            pltpu.SemaphoreType.DMA,
        ],
        compiler_params=pltpu.CompilerParams(use_tc_tiling_on_sc=False),
    )
    return run(x_r, w_word, w_pos)


def kernel(x, W_word, W_pos):
    x_r = x.astype(jnp.int32).reshape(B, 2, HALF)
    pos2 = jnp.tile(W_pos, (CB, 1))
    return _embed(x_r, W_word, pos2).reshape(B, L, DIM)
